# baseline (speedup 1.0000x reference)
import functools

import jax
import jax.numpy as jnp
from jax import lax
from jax.experimental import pallas as pl
from jax.experimental.pallas import tpu as pltpu
from jax.experimental.pallas import tpu_sc as plsc

_CL = 65536
_S = 720896
_E = 999424
_CH = 1024


def _proj_body(w_ref, b_ref, x_ref, o_ref):
    x = x_ref[...]
    w = w_ref[...]
    p = jnp.sum(x * w, axis=0, keepdims=True) + b_ref[0, 0]
    o_ref[...] = p.reshape(o_ref.shape)


def _project_table_tc(table_t, w_col, b):
    D, V = table_t.shape
    main = _S // _CL
    last = (V - 1) // _CL
    G = main + 1

    def in_map(i):
        return (0, jnp.where(i < main, i, last))

    return pl.pallas_call(
        _proj_body,
        grid=(G,),
        in_specs=[
            pl.BlockSpec((D, 1), lambda i: (0, 0)),
            pl.BlockSpec((1, 1), lambda i: (0, 0)),
            pl.BlockSpec((D, _CL), in_map),
        ],
        out_specs=pl.BlockSpec((1, 1, _CL), lambda i: (i, 0, 0)),
        out_shape=jax.ShapeDtypeStruct((G, 1, _CL), jnp.float32),
    )(w_col, b.reshape(1, 1), table_t)


def _w_broadcasts(wb_v, D):
    w_lo = wb_v[pl.ds(0, 16)]
    w_hi = wb_v[pl.ds(16, 16)]
    w_b2 = wb_v[pl.ds(32, 16)]

    def _bcast(vec, lane):
        idx = jnp.full((16,), lane, jnp.int32)
        return vec.at[idx].get(mode="promise_in_bounds")

    wbs = ([_bcast(w_lo, d) for d in range(16)] +
           [_bcast(w_hi, d) for d in range(16)])
    return wbs[:D], _bcast(w_b2, 0)


def _project_table_sc(table_t, wb):
    D, V = table_t.shape
    info = plsc.get_sparse_core_info()
    nc = info.num_cores
    nw = nc * info.num_subcores
    R = _E - _S
    nfull = R // _CH
    tail = R - nfull * _CH
    iters = (nfull + nw - 1) // nw

    mesh = plsc.VectorSubcoreMesh(core_axis_name="c", subcore_axis_name="s")

    @functools.partial(
        pl.kernel,
        mesh=mesh,
        out_type=jax.ShapeDtypeStruct((R,), jnp.float32),
        scratch_types=[
            pltpu.VMEM((D, _CH), jnp.float32),
            pltpu.VMEM((D, _CH), jnp.float32),
            pltpu.VMEM((_CH,), jnp.float32),
            pltpu.VMEM((_CH,), jnp.float32),
            pltpu.VMEM((128,), jnp.float32),
            pltpu.SemaphoreType.DMA,
            pltpu.SemaphoreType.DMA,
        ],
        compiler_params=pltpu.CompilerParams(needs_layout_passes=False),
    )
    def sc_proj_k(tab_hbm, wb_hbm, out_hbm,
                  buf_a, buf_b, ov_a, ov_b, wb_v, sem_a, sem_b):
        wid = lax.axis_index("s") * nc + lax.axis_index("c")
        pltpu.sync_copy(wb_hbm, wb_v)
        wbs, bias = _w_broadcasts(wb_v, D)
        bufs = [buf_a, buf_b]
        ovs = [ov_a, ov_b]
        sems = [sem_a, sem_b]

        def fire(g, b):
            return pltpu.async_copy(
                tab_hbm.at[pl.ds(0, D), pl.ds(_S + g * _CH, _CH)],
                bufs[b], sems[b])

        def wait(b):
            pltpu.make_async_copy(
                tab_hbm.at[pl.ds(0, D), pl.ds(0, _CH)], bufs[b], sems[b]).wait()

        def compute(b, n):
            def group(g, _):
                acc = bias
                for d in range(D):
                    acc = acc + bufs[b][d, pl.ds(g * 16, 16)] * wbs[d]
                ovs[b][pl.ds(g * 16, 16)] = acc
                return 0
            lax.fori_loop(0, n // 16, group, 0)

        @pl.when(wid < nfull)
        def _():
            fire(wid, 0)

        for k in range(iters):
            b = k % 2
            g = wid + k * nw
            gn = wid + (k + 1) * nw

            @pl.when(gn < nfull)
            def _(b=b, gn=gn):
                fire(gn, 1 - b)

            @pl.when(g < nfull)
            def _(b=b, g=g):
                wait(b)
                compute(b, _CH)
                pltpu.sync_copy(ovs[b], out_hbm.at[pl.ds(g * _CH, _CH)])

        if tail:
            @pl.when(wid == nw - 1)
            def _():
                b = iters % 2
                pltpu.sync_copy(
                    tab_hbm.at[pl.ds(0, D), pl.ds(_S + nfull * _CH, tail)],
                    bufs[b].at[pl.ds(0, D), pl.ds(0, tail)])
                compute(b, tail)
                pltpu.sync_copy(ovs[b].at[pl.ds(0, tail)],
                                out_hbm.at[pl.ds(nfull * _CH, tail)])

    return sc_proj_k(table_t, wb)


def _gather_scalars(proj_tc_flat, proj_sc, ids_flat):
    info = plsc.get_sparse_core_info()
    nc, ns = info.num_cores, info.num_subcores
    nw = nc * ns
    B = ids_flat.shape[0]
    assert B % nw == 0
    bpw = B // nw
    R = proj_sc.shape[0]
    z = _S // ns
    qs = R // ns
    vtail = 1024
    tc_tail_src = _S + (_E - (999999 // _CL) * _CL)
    P = _E + vtail

    mesh = plsc.VectorSubcoreMesh(core_axis_name="c", subcore_axis_name="s")

    @functools.partial(
        pl.kernel,
        mesh=mesh,
        out_type=jax.ShapeDtypeStruct((B,), jnp.float32),
        scratch_types=[
            pltpu.VMEM((bpw,), jnp.int32),
            pltpu.VMEM((bpw,), jnp.float32),
            pltpu.VMEM_SHARED((P,), jnp.float32),
            pltpu.SemaphoreType.DMA,
            pltpu.SemaphoreType.DMA,
        ],
    )
    def gather_k(ptc_hbm, psc_hbm, idx_hbm, out_hbm,
                 idx_v, val_v, shared, sem_a, sem_b):
        wid = lax.axis_index("s") * nc + lax.axis_index("c")
        sid = lax.axis_index("s")
        base = wid * bpw
        c1 = pltpu.async_copy(ptc_hbm.at[pl.ds(sid * z, z)],
                              shared.at[pl.ds(sid * z, z)], sem_a)
        c2 = pltpu.async_copy(psc_hbm.at[pl.ds(sid * qs, qs)],
                              shared.at[pl.ds(_S + sid * qs, qs)], sem_b)

        @pl.when(sid == 0)
        def _():
            pltpu.sync_copy(ptc_hbm.at[pl.ds(tc_tail_src, vtail)],
                            shared.at[pl.ds(_E, vtail)])
        pltpu.sync_copy(idx_hbm.at[pl.ds(base, bpw)], idx_v)
        c1.wait()
        c2.wait()
        plsc.subcore_barrier()
        pltpu.sync_copy(shared.at[idx_v], val_v)
        pltpu.sync_copy(val_v, out_hbm.at[pl.ds(base, bpw)])

    return gather_k(proj_tc_flat, proj_sc, ids_flat)


def kernel(input_ids, embed_weight, value_head_weight, value_head_bias):
    B, T = input_ids.shape
    table_t = embed_weight.T
    wb = jnp.zeros((128,), jnp.float32)
    wb = wb.at[:32].set(value_head_weight.reshape(-1))
    wb = wb.at[32].set(value_head_bias[0])
    proj_tc = _project_table_tc(table_t, value_head_weight.reshape(-1, 1),
                                value_head_bias)
    proj_sc = _project_table_sc(table_t, wb)
    ids_flat = input_ids.T.reshape(-1).astype(jnp.int32)
    vals = _gather_scalars(proj_tc.reshape(-1), proj_sc, ids_flat)
    return jnp.transpose(vals.reshape(T, B, 1), (1, 0, 2))


# speedup vs baseline: 1.0101x; 1.0101x over previous
import functools

import jax
import jax.numpy as jnp
from jax import lax
from jax.experimental import pallas as pl
from jax.experimental.pallas import tpu as pltpu
from jax.experimental.pallas import tpu_sc as plsc

_CL = 65536


def _proj_body(w_ref, b_ref, x_ref, o_ref):
    x = x_ref[...]
    w = w_ref[...]
    p = jnp.sum(x * w, axis=0, keepdims=True) + b_ref[0, 0]
    o_ref[...] = p.reshape(o_ref.shape)


def _project_table_tc(table_t, w_col, b):
    D, V = table_t.shape
    G = (V + _CL - 1) // _CL
    return pl.pallas_call(
        _proj_body,
        grid=(G,),
        in_specs=[
            pl.BlockSpec((D, 1), lambda i: (0, 0)),
            pl.BlockSpec((1, 1), lambda i: (0, 0)),
            pl.BlockSpec((D, _CL), lambda i: (0, i)),
        ],
        out_specs=pl.BlockSpec((1, 1, _CL), lambda i: (i, 0, 0)),
        out_shape=jax.ShapeDtypeStruct((G, 1, _CL), jnp.float32),
    )(w_col, b.reshape(1, 1), table_t)


def _gather_scalars(proj_flat, ids_flat):
    info = plsc.get_sparse_core_info()
    nc, ns = info.num_cores, info.num_subcores
    nw = nc * ns
    B = ids_flat.shape[0]
    assert B % nw == 0
    bpw = B // nw

    mesh = plsc.VectorSubcoreMesh(core_axis_name="c", subcore_axis_name="s")

    P = proj_flat.shape[0]
    share = P // ns

    @functools.partial(
        pl.kernel,
        mesh=mesh,
        out_type=jax.ShapeDtypeStruct((B,), jnp.float32),
        scratch_types=[
            pltpu.VMEM((bpw,), jnp.int32),
            pltpu.VMEM((bpw,), jnp.float32),
            pltpu.VMEM_SHARED((P,), jnp.float32),
            pltpu.SemaphoreType.DMA,
        ],
    )
    def gather_k(proj_hbm, idx_hbm, out_hbm, idx_v, val_v, shared, sem):
        wid = lax.axis_index("s") * nc + lax.axis_index("c")
        sid = lax.axis_index("s")
        base = wid * bpw
        cp = pltpu.async_copy(proj_hbm.at[pl.ds(sid * share, share)],
                              shared.at[pl.ds(sid * share, share)], sem)
        pltpu.sync_copy(idx_hbm.at[pl.ds(base, bpw)], idx_v)
        cp.wait()
        plsc.subcore_barrier()
        pltpu.sync_copy(shared.at[idx_v], val_v)
        pltpu.sync_copy(val_v, out_hbm.at[pl.ds(base, bpw)])

    return gather_k(proj_flat, ids_flat)


def kernel(input_ids, embed_weight, value_head_weight, value_head_bias):
    B, T = input_ids.shape
    proj = _project_table_tc(embed_weight.T, value_head_weight.reshape(-1, 1),
                             value_head_bias)
    ids_flat = input_ids.T.reshape(-1).astype(jnp.int32)
    vals = _gather_scalars(proj.reshape(-1), ids_flat)
    return jnp.transpose(vals.reshape(T, B, 1), (1, 0, 2))
